# baseline (speedup 1.0000x reference)
import functools

import jax
import jax.numpy as jnp
from jax import lax
from jax.experimental import pallas as pl
from jax.experimental.pallas import tpu as pltpu
from jax.experimental.pallas import tpu_sc as plsc

NC = 2
NS = 16
NW = NC * NS
CHUNK = 128
PROP_F0 = 0.5
DEG_W = 128


def _ceil_to(a: int, m: int) -> int:
    return (a + m - 1) // m * m


def _deg_sc(dstp, ones_rows, zeros_deg, n: int, rpt: int, bounce: int):
    e_pad = dstp.shape[0]
    epw = e_pad // NW
    n_chunks = epw // CHUNK
    mesh = plsc.VectorSubcoreMesh(core_axis_name="c", subcore_axis_name="s",
                                  num_cores=NC, num_subcores=NS)

    @functools.partial(
        pl.kernel,
        out_type=jax.ShapeDtypeStruct((NC, n, DEG_W), jnp.float32),
        mesh=mesh,
        scratch_types=[
            pltpu.VMEM_SHARED((n, DEG_W), jnp.float32),
            pltpu.VMEM((CHUNK, DEG_W), jnp.float32),
            pltpu.VMEM((bounce, DEG_W), jnp.float32),
            pltpu.VMEM((CHUNK,), jnp.int32),
        ],
    )
    def k(dst_hbm, ones_hbm, zeros_hbm, deg_hbm, degtab, obuf, dbuf, didx):
        core = lax.axis_index("c")
        sub = lax.axis_index("s")
        wid = core * NS + sub

        pltpu.sync_copy(zeros_hbm, dbuf)
        pltpu.sync_copy(ones_hbm, obuf)
        for j in range(rpt // bounce):
            pltpu.sync_copy(dbuf, degtab.at[pl.ds(sub * rpt + j * bounce, bounce)])

        plsc.subcore_barrier()

        def step(i, _):
            pltpu.sync_copy(dst_hbm.at[pl.ds(wid * epw + i * CHUNK, CHUNK)], didx)
            pltpu.sync_copy(obuf, degtab.at[didx], add=True)
            return 0

        lax.fori_loop(0, n_chunks, step, 0)

        plsc.subcore_barrier()

        for j in range(rpt // bounce):
            r = sub * rpt + j * bounce
            pltpu.sync_copy(degtab.at[pl.ds(r, bounce)], dbuf)
            pltpu.sync_copy(dbuf, deg_hbm.at[core, pl.ds(r, bounce)])

    return k(dstp, ones_rows, zeros_deg)


def _prop_sc(g, srcp, dstp, zeros_rows, n: int, rpt: int, bounce: int,
             c0: int, c1: int):
    d = g.shape[1]
    mesh = plsc.VectorSubcoreMesh(core_axis_name="c", subcore_axis_name="s",
                                  num_cores=NC, num_subcores=NS)

    @functools.partial(
        pl.kernel,
        out_type=jax.ShapeDtypeStruct((NC, n, d), jnp.float32),
        mesh=mesh,
        scratch_types=[
            pltpu.VMEM_SHARED((n, d), jnp.float32),
            pltpu.VMEM((CHUNK, d), jnp.float32),
            pltpu.VMEM((CHUNK,), jnp.int32),
            pltpu.VMEM((CHUNK,), jnp.int32),
            pltpu.VMEM((CHUNK,), jnp.int32),
            pltpu.VMEM((CHUNK,), jnp.int32),
            pltpu.SemaphoreType.DMA,
            pltpu.SemaphoreType.DMA,
            pltpu.SemaphoreType.DMA,
        ],
    )
    def k(g_hbm, src_hbm, dst_hbm, zeros_hbm, out, acc, rows,
          sidx0, didx0, sidx1, didx1, sem, isem0, isem1):
        core = lax.axis_index("c")
        sub = lax.axis_index("s")

        pltpu.sync_copy(zeros_hbm, rows)
        for j in range(rpt // bounce):
            pltpu.sync_copy(rows, acc.at[pl.ds(sub * rpt + j * bounce, bounce)])

        plsc.subcore_barrier()

        chunk_base = jnp.where(core == 0, sub * c0, NS * c0 + sub * c1)
        n_my = jnp.where(core == 0, c0, c1)

        def idx_load(i, sidx, didx, sem):
            eb = (chunk_base + jnp.minimum(i, n_my - 1)) * CHUNK
            pltpu.async_copy(src_hbm.at[pl.ds(eb, CHUNK)], sidx, sem)
            pltpu.async_copy(dst_hbm.at[pl.ds(eb, CHUNK)], didx, sem)

        def idx_wait(sidx, didx, sem):
            pltpu.make_async_copy(src_hbm.at[pl.ds(0, CHUNK)], sidx, sem).wait()
            pltpu.make_async_copy(dst_hbm.at[pl.ds(0, CHUNK)], didx, sem).wait()

        idx_load(0, sidx0, didx0, isem0)

        def pair(j, _):
            i0 = 2 * j
            idx_load(i0 + 1, sidx1, didx1, isem1)
            idx_wait(sidx0, didx0, isem0)
            pltpu.async_copy(g_hbm.at[sidx0], rows, sem).wait()
            pltpu.sync_copy(rows, acc.at[didx0], add=True)
            idx_load(i0 + 2, sidx0, didx0, isem0)
            idx_wait(sidx1, didx1, isem1)
            pltpu.async_copy(g_hbm.at[sidx1], rows, sem).wait()
            pltpu.sync_copy(rows, acc.at[didx1], add=True)
            return 0

        lax.fori_loop(0, n_my // 2, pair, 0)
        idx_wait(sidx0, didx0, isem0)

        plsc.subcore_barrier()

        for j in range(rpt // bounce):
            r = sub * rpt + j * bounce
            pltpu.sync_copy(acc.at[pl.ds(r, bounce)], rows)
            pltpu.sync_copy(rows, out.at[core, pl.ds(r, bounce)])

    return k(g, srcp, dstp, zeros_rows)


def _dis(deg_blk):
    return lax.rsqrt(deg_blk[0, :, 0:1] + deg_blk[1, :, 0:1] + 1.0)


def _mm_scale_tc(x, w, deg, br: int):
    n, din = x.shape
    dout = w.shape[1]

    def body(x_ref, w_ref, deg_ref, o_ref):
        h = jnp.dot(x_ref[...], w_ref[...], preferred_element_type=jnp.float32)
        o_ref[...] = _dis(deg_ref[...]) * h

    return pl.pallas_call(
        body,
        grid=(n // br,),
        in_specs=[
            pl.BlockSpec((br, din), lambda i: (i, 0)),
            pl.BlockSpec((din, dout), lambda i: (0, 0)),
            pl.BlockSpec((NC, br, DEG_W), lambda i: (0, i, 0)),
        ],
        out_specs=pl.BlockSpec((br, dout), lambda i: (i, 0)),
        out_shape=jax.ShapeDtypeStruct((n, dout), jnp.float32),
    )(x, w, deg)


def _mid_tc(accs, g1, b1, wcat, deg, br: int):
    _, n, d = accs.shape
    dout = wcat.shape[1]

    def body(a_ref, g_ref, b_ref, w_ref, deg_ref, o_ref):
        dis = _dis(deg_ref[...])
        asum = a_ref[0] + a_ref[1] + g_ref[...]
        h = jnp.maximum(dis * asum + b_ref[0:1, :], 0.0)
        o_ref[...] = dis * jnp.dot(h, w_ref[...], preferred_element_type=jnp.float32)

    return pl.pallas_call(
        body,
        grid=(n // br,),
        in_specs=[
            pl.BlockSpec((NC, br, d), lambda i: (0, i, 0)),
            pl.BlockSpec((br, d), lambda i: (i, 0)),
            pl.BlockSpec((8, d), lambda i: (0, 0)),
            pl.BlockSpec((d, dout), lambda i: (0, 0)),
            pl.BlockSpec((NC, br, DEG_W), lambda i: (0, i, 0)),
        ],
        out_specs=pl.BlockSpec((br, dout), lambda i: (i, 0)),
        out_shape=jax.ShapeDtypeStruct((n, dout), jnp.float32),
    )(accs, g1, b1, wcat, deg)


def _final_tc(accs, g2, bcat, deg, br: int):
    _, n, d = accs.shape

    def body(a_ref, g_ref, b_ref, deg_ref, o_ref):
        dis = _dis(deg_ref[...])
        o_ref[...] = dis * (a_ref[0] + a_ref[1] + g_ref[...]) + b_ref[0:1, :]

    return pl.pallas_call(
        body,
        grid=(n // br,),
        in_specs=[
            pl.BlockSpec((NC, br, d), lambda i: (0, i, 0)),
            pl.BlockSpec((br, d), lambda i: (i, 0)),
            pl.BlockSpec((8, d), lambda i: (0, 0)),
            pl.BlockSpec((NC, br, DEG_W), lambda i: (0, i, 0)),
        ],
        out_specs=pl.BlockSpec((br, d), lambda i: (i, 0)),
        out_shape=jax.ShapeDtypeStruct((n, d), jnp.float32),
    )(accs, g2, bcat, deg)


def kernel(x, edge_index, W1, b1, Wmu, bmu, Wlv, blv):
    n, din = x.shape
    e = edge_index.shape[1]
    hid = W1.shape[1]
    z = Wmu.shape[1]

    npad = _ceil_to(n, NS * CHUNK)
    rpt = npad // NS
    bounce = CHUNK
    br = 512
    xp = jnp.pad(x, ((0, npad - n), (0, 0)))

    e_pad = _ceil_to(e, NW * CHUNK * 2)
    c_tot = e_pad // CHUNK
    cpp = c_tot // NS
    c0 = max(2, 2 * int(round(cpp * PROP_F0 / 2)))
    c1 = cpp - c0
    src = edge_index[0]
    dst = edge_index[1]
    pad = e_pad - e
    srcp = jnp.concatenate([src, jnp.zeros((pad,), jnp.int32)])
    dstp = jnp.concatenate([dst, jnp.full((pad,), n, jnp.int32)])

    ones_rows = jnp.ones((CHUNK, DEG_W), jnp.float32)
    zeros_rows = jnp.zeros((bounce, hid), jnp.float32)

    deg = _deg_sc(dstp, ones_rows, zeros_rows, npad, rpt, bounce)

    g1 = _mm_scale_tc(xp, W1, deg, br)
    acc1 = _prop_sc(g1, srcp, dstp, zeros_rows, npad, rpt, bounce, c0, c1)

    wcat = jnp.concatenate([Wmu, Wlv], axis=1)
    bcat = jnp.broadcast_to(jnp.concatenate([bmu, blv])[None, :], (8, 2 * z))
    b1_b = jnp.broadcast_to(b1[None, :], (8, hid))

    g2 = _mid_tc(acc1, g1, b1_b, wcat, deg, br)
    acc2 = _prop_sc(g2, srcp, dstp, zeros_rows, npad, rpt, bounce, c0, c1)

    out = _final_tc(acc2, g2, bcat, deg, br)
    return (out[:n, :z], out[:n, z:])


# speedup vs baseline: 1.2418x; 1.2418x over previous
import functools

import jax
import jax.numpy as jnp
from jax import lax
from jax.experimental import pallas as pl
from jax.experimental.pallas import tpu as pltpu
from jax.experimental.pallas import tpu_sc as plsc

NC = 2
NS = 16
NW = NC * NS
CHUNK = 128
PROP_F0 = 0.5
DEG_W = 128


def _ceil_to(a: int, m: int) -> int:
    return (a + m - 1) // m * m


def _deg_sc(dstp, ones_rows, zeros_deg, n: int, rpt: int, bounce: int):
    e_pad = dstp.shape[0]
    epw = e_pad // NW
    n_chunks = epw // CHUNK
    mesh = plsc.VectorSubcoreMesh(core_axis_name="c", subcore_axis_name="s",
                                  num_cores=NC, num_subcores=NS)

    @functools.partial(
        pl.kernel,
        out_type=jax.ShapeDtypeStruct((NC, n, DEG_W), jnp.float32),
        mesh=mesh,
        scratch_types=[
            pltpu.VMEM_SHARED((n, DEG_W), jnp.float32),
            pltpu.VMEM((CHUNK, DEG_W), jnp.float32),
            pltpu.VMEM((bounce, DEG_W), jnp.float32),
            pltpu.VMEM((CHUNK,), jnp.int32),
        ],
    )
    def k(dst_hbm, ones_hbm, zeros_hbm, deg_hbm, degtab, obuf, dbuf, didx):
        core = lax.axis_index("c")
        sub = lax.axis_index("s")
        wid = core * NS + sub

        pltpu.sync_copy(zeros_hbm, dbuf)
        pltpu.sync_copy(ones_hbm, obuf)
        for j in range(rpt // bounce):
            pltpu.sync_copy(dbuf, degtab.at[pl.ds(sub * rpt + j * bounce, bounce)])

        plsc.subcore_barrier()

        def step(i, _):
            pltpu.sync_copy(dst_hbm.at[pl.ds(wid * epw + i * CHUNK, CHUNK)], didx)
            pltpu.sync_copy(obuf, degtab.at[didx], add=True)
            return 0

        lax.fori_loop(0, n_chunks, step, 0)

        plsc.subcore_barrier()

        for j in range(rpt // bounce):
            r = sub * rpt + j * bounce
            pltpu.sync_copy(degtab.at[pl.ds(r, bounce)], dbuf)
            pltpu.sync_copy(dbuf, deg_hbm.at[core, pl.ds(r, bounce)])

    return k(dstp, ones_rows, zeros_deg)


def _prop_sc(g, srcp, dstp, zeros_rows, n: int, rpt: int, bounce: int,
             c0: int, c1: int):
    d = g.shape[1]
    mesh = plsc.VectorSubcoreMesh(core_axis_name="c", subcore_axis_name="s",
                                  num_cores=NC, num_subcores=NS)

    @functools.partial(
        pl.kernel,
        out_type=jax.ShapeDtypeStruct((NC, n, d), jnp.float32),
        mesh=mesh,
        scratch_types=[
            pltpu.VMEM_SHARED((n, d), jnp.float32),
            pltpu.VMEM((CHUNK, d), jnp.float32),
            pltpu.VMEM((CHUNK,), jnp.int32),
            pltpu.VMEM((CHUNK,), jnp.int32),
            pltpu.SemaphoreType.DMA,
        ],
    )
    def k(g_hbm, src_hbm, dst_hbm, zeros_hbm, out, acc, rows,
          sidx, didx, sem):
        core = lax.axis_index("c")
        sub = lax.axis_index("s")

        pltpu.sync_copy(zeros_hbm, rows)
        for j in range(rpt // bounce):
            pltpu.sync_copy(rows, acc.at[pl.ds(sub * rpt + j * bounce, bounce)])

        plsc.subcore_barrier()

        chunk_base = jnp.where(core == 0, sub * c0, NS * c0 + sub * c1)
        n_my = jnp.where(core == 0, c0, c1)

        def step(i, _):
            eb = (chunk_base + i) * CHUNK
            pltpu.sync_copy(src_hbm.at[pl.ds(eb, CHUNK)], sidx)
            pltpu.sync_copy(dst_hbm.at[pl.ds(eb, CHUNK)], didx)
            pltpu.async_copy(g_hbm.at[sidx], rows, sem).wait()
            pltpu.sync_copy(rows, acc.at[didx], add=True)
            return 0

        lax.fori_loop(0, n_my, step, 0)

        plsc.subcore_barrier()

        for j in range(rpt // bounce):
            r = sub * rpt + j * bounce
            pltpu.sync_copy(acc.at[pl.ds(r, bounce)], rows)
            pltpu.sync_copy(rows, out.at[core, pl.ds(r, bounce)])

    return k(g, srcp, dstp, zeros_rows)


def _dis(deg_blk):
    return lax.rsqrt(deg_blk[0, :, 0:1] + deg_blk[1, :, 0:1] + 1.0)


def _mm_tc(x, w, br: int):
    n, din = x.shape
    dout = w.shape[1]

    def body(x_ref, w_ref, o_ref):
        o_ref[...] = jnp.dot(x_ref[...], w_ref[...],
                             preferred_element_type=jnp.float32)

    return pl.pallas_call(
        body,
        grid=(n // br,),
        in_specs=[
            pl.BlockSpec((br, din), lambda i: (i, 0)),
            pl.BlockSpec((din, dout), lambda i: (0, 0)),
        ],
        out_specs=pl.BlockSpec((br, dout), lambda i: (i, 0)),
        out_shape=jax.ShapeDtypeStruct((n, dout), jnp.float32),
    )(x, w)


def _scale_tc(u, deg, br: int):
    n, d = u.shape

    def body(u_ref, deg_ref, o_ref):
        o_ref[...] = _dis(deg_ref[...]) * u_ref[...]

    return pl.pallas_call(
        body,
        grid=(n // br,),
        in_specs=[
            pl.BlockSpec((br, d), lambda i: (i, 0)),
            pl.BlockSpec((NC, br, DEG_W), lambda i: (0, i, 0)),
        ],
        out_specs=pl.BlockSpec((br, d), lambda i: (i, 0)),
        out_shape=jax.ShapeDtypeStruct((n, d), jnp.float32),
    )(u, deg)


def _mid_tc(accs, g1, b1, wcat, deg, br: int):
    _, n, d = accs.shape
    dout = wcat.shape[1]

    def body(a_ref, g_ref, b_ref, w_ref, deg_ref, o_ref):
        dis = _dis(deg_ref[...])
        asum = a_ref[0] + a_ref[1] + g_ref[...]
        h = jnp.maximum(dis * asum + b_ref[0:1, :], 0.0)
        o_ref[...] = dis * jnp.dot(h, w_ref[...], preferred_element_type=jnp.float32)

    return pl.pallas_call(
        body,
        grid=(n // br,),
        in_specs=[
            pl.BlockSpec((NC, br, d), lambda i: (0, i, 0)),
            pl.BlockSpec((br, d), lambda i: (i, 0)),
            pl.BlockSpec((8, d), lambda i: (0, 0)),
            pl.BlockSpec((d, dout), lambda i: (0, 0)),
            pl.BlockSpec((NC, br, DEG_W), lambda i: (0, i, 0)),
        ],
        out_specs=pl.BlockSpec((br, dout), lambda i: (i, 0)),
        out_shape=jax.ShapeDtypeStruct((n, dout), jnp.float32),
    )(accs, g1, b1, wcat, deg)


def _final_tc(accs, g2, bcat, deg, br: int):
    _, n, d = accs.shape

    def body(a_ref, g_ref, b_ref, deg_ref, o_ref):
        dis = _dis(deg_ref[...])
        o_ref[...] = dis * (a_ref[0] + a_ref[1] + g_ref[...]) + b_ref[0:1, :]

    return pl.pallas_call(
        body,
        grid=(n // br,),
        in_specs=[
            pl.BlockSpec((NC, br, d), lambda i: (0, i, 0)),
            pl.BlockSpec((br, d), lambda i: (i, 0)),
            pl.BlockSpec((8, d), lambda i: (0, 0)),
            pl.BlockSpec((NC, br, DEG_W), lambda i: (0, i, 0)),
        ],
        out_specs=pl.BlockSpec((br, d), lambda i: (i, 0)),
        out_shape=jax.ShapeDtypeStruct((n, d), jnp.float32),
    )(accs, g2, bcat, deg)


def kernel(x, edge_index, W1, b1, Wmu, bmu, Wlv, blv):
    n, din = x.shape
    e = edge_index.shape[1]
    hid = W1.shape[1]
    z = Wmu.shape[1]

    npad = _ceil_to(n, NS * CHUNK)
    rpt = npad // NS
    bounce = CHUNK
    br = 512
    xp = jnp.pad(x, ((0, npad - n), (0, 0)))

    e_pad = _ceil_to(e, NW * CHUNK)
    c_tot = e_pad // CHUNK
    cpp = c_tot // NS
    c0 = max(1, int(round(cpp * PROP_F0)))
    c1 = cpp - c0
    src = edge_index[0]
    dst = edge_index[1]
    pad = e_pad - e
    srcp = jnp.concatenate([src, jnp.zeros((pad,), jnp.int32)])
    dstp = jnp.concatenate([dst, jnp.full((pad,), n, jnp.int32)])

    ones_rows = jnp.ones((CHUNK, DEG_W), jnp.float32)
    zeros_rows = jnp.zeros((bounce, hid), jnp.float32)

    u1 = _mm_tc(xp, W1, br)
    deg = _deg_sc(dstp, ones_rows, zeros_rows, npad, rpt, bounce)
    g1 = _scale_tc(u1, deg, br)
    acc1 = _prop_sc(g1, srcp, dstp, zeros_rows, npad, rpt, bounce, c0, c1)

    wcat = jnp.concatenate([Wmu, Wlv], axis=1)
    bcat = jnp.broadcast_to(jnp.concatenate([bmu, blv])[None, :], (8, 2 * z))
    b1_b = jnp.broadcast_to(b1[None, :], (8, hid))

    g2 = _mid_tc(acc1, g1, b1_b, wcat, deg, br)
    acc2 = _prop_sc(g2, srcp, dstp, zeros_rows, npad, rpt, bounce, c0, c1)

    out = _final_tc(acc2, g2, bcat, deg, br)
    return (out[:n, :z], out[:n, z:])


# speedup vs baseline: 1.2477x; 1.0047x over previous
import functools

import jax
import jax.numpy as jnp
from jax import lax
from jax.experimental import pallas as pl
from jax.experimental.pallas import tpu as pltpu
from jax.experimental.pallas import tpu_sc as plsc

NC = 2
NS = 16
NW = NC * NS
CHUNK = 128
PROP_F0 = 0.5
DEG_W = 128


def _ceil_to(a: int, m: int) -> int:
    return (a + m - 1) // m * m


def _deg_sc(dstp, ones_rows, zeros_deg, n: int, rpt: int, bounce: int):
    e_pad = dstp.shape[0]
    epw = e_pad // NW
    n_chunks = epw // CHUNK
    mesh = plsc.VectorSubcoreMesh(core_axis_name="c", subcore_axis_name="s",
                                  num_cores=NC, num_subcores=NS)

    @functools.partial(
        pl.kernel,
        out_type=jax.ShapeDtypeStruct((NC, n, DEG_W), jnp.float32),
        mesh=mesh,
        scratch_types=[
            pltpu.VMEM_SHARED((n, DEG_W), jnp.float32),
            pltpu.VMEM((CHUNK, DEG_W), jnp.float32),
            pltpu.VMEM((bounce, DEG_W), jnp.float32),
            pltpu.VMEM((CHUNK,), jnp.int32),
        ],
    )
    def k(dst_hbm, ones_hbm, zeros_hbm, deg_hbm, degtab, obuf, dbuf, didx):
        core = lax.axis_index("c")
        sub = lax.axis_index("s")
        wid = core * NS + sub

        pltpu.sync_copy(zeros_hbm, dbuf)
        pltpu.sync_copy(ones_hbm, obuf)
        for j in range(rpt // bounce):
            pltpu.sync_copy(dbuf, degtab.at[pl.ds(sub * rpt + j * bounce, bounce)])

        plsc.subcore_barrier()

        def step(i, _):
            pltpu.sync_copy(dst_hbm.at[pl.ds(wid * epw + i * CHUNK, CHUNK)], didx)
            pltpu.sync_copy(obuf, degtab.at[didx], add=True)
            return 0

        lax.fori_loop(0, n_chunks, step, 0)

        plsc.subcore_barrier()

        for j in range(rpt // bounce):
            r = sub * rpt + j * bounce
            pltpu.sync_copy(degtab.at[pl.ds(r, bounce)], dbuf)
            pltpu.sync_copy(dbuf, deg_hbm.at[core, pl.ds(r, bounce)])

    return k(dstp, ones_rows, zeros_deg)


def _prop_sc(g, srcp, dstp, zeros_rows, n: int, rpt: int, bounce: int,
             c0: int, c1: int):
    d = g.shape[1]
    mesh = plsc.VectorSubcoreMesh(core_axis_name="c", subcore_axis_name="s",
                                  num_cores=NC, num_subcores=NS)

    @functools.partial(
        pl.kernel,
        out_type=jax.ShapeDtypeStruct((NC, n, d), jnp.float32),
        mesh=mesh,
        scratch_types=[
            pltpu.VMEM_SHARED((n, d), jnp.float32),
            pltpu.VMEM((CHUNK, d), jnp.float32),
            pltpu.VMEM((CHUNK,), jnp.int32),
            pltpu.VMEM((CHUNK,), jnp.int32),
            pltpu.SemaphoreType.DMA,
        ],
    )
    def k(g_hbm, src_hbm, dst_hbm, zeros_hbm, out, acc, rows,
          sidx, didx, sem):
        core = lax.axis_index("c")
        sub = lax.axis_index("s")

        pltpu.sync_copy(zeros_hbm, rows)
        for j in range(rpt // bounce):
            pltpu.sync_copy(rows, acc.at[pl.ds(sub * rpt + j * bounce, bounce)])

        plsc.subcore_barrier()

        chunk_base = jnp.where(core == 0, sub * c0, NS * c0 + sub * c1)
        n_my = jnp.where(core == 0, c0, c1)

        def step(i, _):
            eb = (chunk_base + i) * CHUNK
            pltpu.sync_copy(src_hbm.at[pl.ds(eb, CHUNK)], sidx)
            pltpu.sync_copy(dst_hbm.at[pl.ds(eb, CHUNK)], didx)
            pltpu.async_copy(g_hbm.at[sidx], rows, sem).wait()
            pltpu.sync_copy(rows, acc.at[didx], add=True)
            return 0

        lax.fori_loop(0, n_my, step, 0)

        plsc.subcore_barrier()

        for j in range(rpt // bounce):
            r = sub * rpt + j * bounce
            pltpu.sync_copy(acc.at[pl.ds(r, bounce)], rows)
            pltpu.sync_copy(rows, out.at[core, pl.ds(r, bounce)])

    return k(g, srcp, dstp, zeros_rows)


def _dis(deg_blk):
    return lax.rsqrt(deg_blk[0, :, 0:1] + deg_blk[1, :, 0:1] + 1.0)


def _mm_scale_tc(x, w, deg, br: int):
    n, din = x.shape
    dout = w.shape[1]

    def body(x_ref, w_ref, deg_ref, o_ref):
        h = jnp.dot(x_ref[...], w_ref[...], preferred_element_type=jnp.float32)
        o_ref[...] = _dis(deg_ref[...]) * h

    return pl.pallas_call(
        body,
        grid=(n // br,),
        in_specs=[
            pl.BlockSpec((br, din), lambda i: (i, 0)),
            pl.BlockSpec((din, dout), lambda i: (0, 0)),
            pl.BlockSpec((NC, br, DEG_W), lambda i: (0, i, 0)),
        ],
        out_specs=pl.BlockSpec((br, dout), lambda i: (i, 0)),
        out_shape=jax.ShapeDtypeStruct((n, dout), jnp.float32),
    )(x, w, deg)


def _mid_tc(accs, g1, b1, wcat, deg, br: int):
    _, n, d = accs.shape
    dout = wcat.shape[1]

    def body(a_ref, g_ref, b_ref, w_ref, deg_ref, o_ref):
        dis = _dis(deg_ref[...])
        asum = a_ref[0] + a_ref[1] + g_ref[...]
        h = jnp.maximum(dis * asum + b_ref[0:1, :], 0.0)
        o_ref[...] = dis * jnp.dot(h, w_ref[...], preferred_element_type=jnp.float32)

    return pl.pallas_call(
        body,
        grid=(n // br,),
        in_specs=[
            pl.BlockSpec((NC, br, d), lambda i: (0, i, 0)),
            pl.BlockSpec((br, d), lambda i: (i, 0)),
            pl.BlockSpec((8, d), lambda i: (0, 0)),
            pl.BlockSpec((d, dout), lambda i: (0, 0)),
            pl.BlockSpec((NC, br, DEG_W), lambda i: (0, i, 0)),
        ],
        out_specs=pl.BlockSpec((br, dout), lambda i: (i, 0)),
        out_shape=jax.ShapeDtypeStruct((n, dout), jnp.float32),
    )(accs, g1, b1, wcat, deg)


def _final_tc(accs, g2, bcat, deg, br: int):
    _, n, d = accs.shape

    def body(a_ref, g_ref, b_ref, deg_ref, o_ref):
        dis = _dis(deg_ref[...])
        o_ref[...] = dis * (a_ref[0] + a_ref[1] + g_ref[...]) + b_ref[0:1, :]

    return pl.pallas_call(
        body,
        grid=(n // br,),
        in_specs=[
            pl.BlockSpec((NC, br, d), lambda i: (0, i, 0)),
            pl.BlockSpec((br, d), lambda i: (i, 0)),
            pl.BlockSpec((8, d), lambda i: (0, 0)),
            pl.BlockSpec((NC, br, DEG_W), lambda i: (0, i, 0)),
        ],
        out_specs=pl.BlockSpec((br, d), lambda i: (i, 0)),
        out_shape=jax.ShapeDtypeStruct((n, d), jnp.float32),
    )(accs, g2, bcat, deg)


def kernel(x, edge_index, W1, b1, Wmu, bmu, Wlv, blv):
    n, din = x.shape
    e = edge_index.shape[1]
    hid = W1.shape[1]
    z = Wmu.shape[1]

    npad = _ceil_to(n, NS * CHUNK)
    rpt = npad // NS
    bounce = CHUNK
    br = 512
    xp = jnp.pad(x, ((0, npad - n), (0, 0)))

    e_pad = _ceil_to(e, NW * CHUNK)
    c_tot = e_pad // CHUNK
    cpp = c_tot // NS
    c0 = max(1, int(round(cpp * PROP_F0)))
    c1 = cpp - c0
    src = edge_index[0]
    dst = edge_index[1]
    pad = e_pad - e
    srcp = jnp.concatenate([src, jnp.zeros((pad,), jnp.int32)])
    dstp = jnp.concatenate([dst, jnp.full((pad,), n, jnp.int32)])

    ones_rows = jnp.ones((CHUNK, DEG_W), jnp.float32)
    zeros_rows = jnp.zeros((bounce, hid), jnp.float32)

    deg = _deg_sc(dstp, ones_rows, zeros_rows, npad, rpt, bounce)

    g1 = _mm_scale_tc(xp, W1, deg, br)
    acc1 = _prop_sc(g1, srcp, dstp, zeros_rows, npad, rpt, bounce, c0, c1)

    wcat = jnp.concatenate([Wmu, Wlv], axis=1)
    bcat = jnp.broadcast_to(jnp.concatenate([bmu, blv])[None, :], (8, 2 * z))
    b1_b = jnp.broadcast_to(b1[None, :], (8, hid))

    g2 = _mid_tc(acc1, g1, b1_b, wcat, deg, br)
    acc2 = _prop_sc(g2, srcp, dstp, zeros_rows, npad, rpt, bounce, c0, c1)

    out = _final_tc(acc2, g2, bcat, deg, br)
    return (out[:n, :z], out[:n, z:])
